# Initial kernel scaffold; baseline (speedup 1.0000x reference)
import jax
import jax.numpy as jnp
from jax.experimental import pallas as pl


def kernel(x, edge_index, edge_attr, W0, b0, W1, b1, W_out, b_out):
    raise NotImplementedError("write your pallas kernel here")


# speedup vs baseline: 3.6818x; 3.6818x over previous
import functools

import jax
import jax.numpy as jnp
from jax import lax
from jax.experimental import pallas as pl
from jax.experimental.pallas import tpu as pltpu
from jax.experimental.pallas import tpu_sc as plsc

N = 10000
E = 320000
D = 128
NSC = 2
NTILE = 16
NW = NSC * NTILE
EPT = E // NW
K = 80
NCHUNK = EPT // K
NACC = 10240
RPT = NACC // NTILE

_MESH = plsc.VectorSubcoreMesh(core_axis_name="c", subcore_axis_name="s")


def _segsum_rows_body(x_hbm, src_hbm, dst_hbm, zx_hbm, out_hbm,
                      srcv, dstv, rows, accx, sem):
    c = lax.axis_index("c")
    s = lax.axis_index("s")
    r0 = pl.multiple_of(s * RPT, RPT)
    pltpu.sync_copy(zx_hbm.at[pl.ds(r0, RPT)], accx.at[pl.ds(r0, RPT)])
    plsc.subcore_barrier()

    e0 = (c * NTILE + s) * EPT

    def chunk(i, carry):
        base = pl.multiple_of(e0 + i * K, 8)
        pltpu.sync_copy(src_hbm.at[pl.ds(base, K)], srcv)
        pltpu.sync_copy(dst_hbm.at[pl.ds(base, K)], dstv)
        pltpu.async_copy(x_hbm.at[srcv], rows, sem).wait()
        pltpu.sync_copy(rows, accx.at[dstv], add=True)
        return carry

    lax.fori_loop(0, NCHUNK, chunk, 0)
    plsc.subcore_barrier()
    pltpu.sync_copy(accx.at[pl.ds(r0, RPT)], out_hbm.at[c, pl.ds(r0, RPT)])


_sc_segsum_rows = functools.partial(
    pl.kernel,
    out_type=jax.ShapeDtypeStruct((NSC, NACC, D), jnp.float32),
    mesh=_MESH,
    scratch_types=[
        pltpu.VMEM((K,), jnp.int32),
        pltpu.VMEM((K,), jnp.int32),
        pltpu.VMEM((K, D), jnp.float32),
        pltpu.VMEM_SHARED((NACC, D), jnp.float32),
        pltpu.SemaphoreType.DMA,
    ],
    name="sc_segsum_rows")(_segsum_rows_body)


def _segsum_ea_body(ea_hbm, dst_hbm, zx_hbm, out_hbm,
                    dstv, eav, rows, acce, sem):
    c = lax.axis_index("c")
    s = lax.axis_index("s")
    r0 = pl.multiple_of(s * RPT, RPT)
    pltpu.sync_copy(zx_hbm.at[pl.ds(r0, RPT)], acce.at[pl.ds(r0, RPT)])
    pltpu.sync_copy(zx_hbm.at[pl.ds(0, K)], rows)
    plsc.subcore_barrier()

    e0 = (c * NTILE + s) * EPT
    lanes = lax.iota(jnp.int32, 16)
    is_attr = lanes < 4
    one_col = jnp.where(lanes == 4, 1.0, 0.0).astype(jnp.float32)

    def chunk(i, carry):
        base = pl.multiple_of(e0 + i * K, 8)
        pltpu.sync_copy(dst_hbm.at[pl.ds(base, K)], dstv)
        pltpu.sync_copy(ea_hbm.at[pl.ds(base * 4, K * 4)], eav.at[pl.ds(0, K * 4)])
        for e in range(K):
            window = eav[pl.ds(4 * e, 16)]
            rows[e, pl.ds(0, 16)] = jnp.where(is_attr, window, one_col)
        pltpu.sync_copy(rows, acce.at[dstv], add=True)
        return carry

    lax.fori_loop(0, NCHUNK, chunk, 0)
    plsc.subcore_barrier()
    pltpu.sync_copy(acce.at[pl.ds(r0, RPT)], out_hbm.at[c, pl.ds(r0, RPT)])


_sc_segsum_ea = functools.partial(
    pl.kernel,
    out_type=jax.ShapeDtypeStruct((NSC, NACC, D), jnp.float32),
    mesh=_MESH,
    scratch_types=[
        pltpu.VMEM((K,), jnp.int32),
        pltpu.VMEM((K * 4 + 16,), jnp.float32),
        pltpu.VMEM((K, D), jnp.float32),
        pltpu.VMEM_SHARED((NACC, D), jnp.float32),
        pltpu.SemaphoreType.DMA,
    ],
    name="sc_segsum_ea")(_segsum_ea_body)


_RB = 1024
_GRID = NACC // _RB


def _tc_hidden_body(p_ref, q_ref, wx_ref, we_ref, o_ref):
    p = p_ref[0] + p_ref[1]
    q = q_ref[0] + q_ref[1]
    h = jnp.dot(p, wx_ref[...], preferred_element_type=jnp.float32,
                precision=lax.Precision.HIGHEST)
    h = h + jnp.dot(q, we_ref[...], preferred_element_type=jnp.float32,
                    precision=lax.Precision.HIGHEST)
    o_ref[...] = jnp.maximum(h, 0.0)


def _tc_hidden(P, Q, Wx, We):
    return pl.pallas_call(
        _tc_hidden_body,
        grid=(_GRID,),
        in_specs=[pl.BlockSpec((NSC, _RB, D), lambda i: (0, i, 0)),
                  pl.BlockSpec((NSC, _RB, D), lambda i: (0, i, 0)),
                  pl.BlockSpec((D, D), lambda i: (0, 0)),
                  pl.BlockSpec((D, D), lambda i: (0, 0))],
        out_specs=pl.BlockSpec((_RB, D), lambda i: (i, 0)),
        out_shape=jax.ShapeDtypeStruct((NACC, D), jnp.float32),
    )(P, Q, Wx, We)


def _tc_final_body(p_ref, q_ref, wx_ref, we_ref, wo_ref, bo_ref, o_ref):
    p = p_ref[0] + p_ref[1]
    q = q_ref[0] + q_ref[1]
    h = jnp.dot(p, wx_ref[...], preferred_element_type=jnp.float32,
                precision=lax.Precision.HIGHEST)
    h = h + jnp.dot(q, we_ref[...], preferred_element_type=jnp.float32,
                    precision=lax.Precision.HIGHEST)
    h = jnp.maximum(h, 0.0)
    o_ref[...] = jnp.dot(h, wo_ref[...], preferred_element_type=jnp.float32,
                         precision=lax.Precision.HIGHEST) + bo_ref[...]


def _tc_final(P, Q, Wx, We, Wout, bout):
    return pl.pallas_call(
        _tc_final_body,
        grid=(_GRID,),
        in_specs=[pl.BlockSpec((NSC, _RB, D), lambda i: (0, i, 0)),
                  pl.BlockSpec((NSC, _RB, D), lambda i: (0, i, 0)),
                  pl.BlockSpec((D, D), lambda i: (0, 0)),
                  pl.BlockSpec((D, D), lambda i: (0, 0)),
                  pl.BlockSpec((D, D), lambda i: (0, 0)),
                  pl.BlockSpec((1, D), lambda i: (0, 0))],
        out_specs=pl.BlockSpec((_RB, D), lambda i: (i, 0)),
        out_shape=jax.ShapeDtypeStruct((NACC, D), jnp.float32),
    )(P, Q, Wx, We, Wout, bout)


def _pad_we(We4, b):
    return jnp.concatenate(
        [We4, b[None, :], jnp.zeros((D - 5, D), jnp.float32)], axis=0)


def kernel(x, edge_index, edge_attr, W0, b0, W1, b1, W_out, b_out):
    src = edge_index[0].astype(jnp.int32)
    dst = edge_index[1].astype(jnp.int32)
    ea_flat = edge_attr.astype(jnp.float32).reshape(E * 4)
    W0x, W0e = W0[:D], _pad_we(W0[D:], b0)
    W1x, W1e = W1[:D], _pad_we(W1[D:], b1)
    zx = jnp.zeros((NACC, D), jnp.float32)

    P = _sc_segsum_rows(x, src, dst, zx)
    Q = _sc_segsum_ea(ea_flat, dst, zx)
    h1 = _tc_hidden(P, Q, W0x, W0e)
    R = _sc_segsum_rows(h1, src, dst, zx)
    return _tc_final(R, Q, W1x, W1e, W_out, b_out[None, :])[:N]


# speedup vs baseline: 6.2566x; 1.6994x over previous
import functools

import jax
import jax.numpy as jnp
from jax import lax
from jax.experimental import pallas as pl
from jax.experimental.pallas import tpu as pltpu
from jax.experimental.pallas import tpu_sc as plsc

N = 10000
E = 320000
D = 128
NSC = 2
NTILE = 16
NW = NSC * NTILE
EPT = E // NW
K = 80
NCHUNK = EPT // K
PAIRS = NCHUNK // 2
TAIL = NCHUNK - 1
NACC = 10240
RPT = NACC // NTILE

_MESH = plsc.VectorSubcoreMesh(core_axis_name="c", subcore_axis_name="s")


def _segsum_rows_body(x_hbm, src_hbm, dst3_hbm, zx_hbm, out_hbm,
                      sv0, sv1, dst2v, rows0, rows1, accx,
                      is0, is1, gs0, gs1, ss0, ss1):
    c = lax.axis_index("c")
    s = lax.axis_index("s")
    wid = c * NTILE + s
    r0 = pl.multiple_of(s * RPT, RPT)
    pltpu.sync_copy(zx_hbm.at[pl.ds(r0, RPT)], accx.at[pl.ds(r0, RPT)])
    pltpu.sync_copy(dst3_hbm.at[wid], dst2v)
    plsc.subcore_barrier()

    e0 = wid * EPT

    def idxload(i, buf, sem):
        base = pl.multiple_of(e0 + i * K, 8)
        pltpu.async_copy(src_hbm.at[pl.ds(base, K)], buf, sem)

    def wait_idx(i, buf, sem):
        base = pl.multiple_of(e0 + i * K, 8)
        pltpu.make_async_copy(src_hbm.at[pl.ds(base, K)], buf, sem).wait()

    def gather(buf_idx, buf, sem):
        pltpu.async_copy(x_hbm.at[buf_idx], buf, sem)

    def wait_gather(buf_idx, buf, sem):
        pltpu.make_async_copy(x_hbm.at[buf_idx], buf, sem).wait()

    def scatter(i, buf, sem):
        pltpu.async_copy(buf, accx.at[dst2v.at[i]], sem, add=True)

    def wait_scatter(i, buf, sem):
        pltpu.make_async_copy(buf, accx.at[dst2v.at[i]], sem).wait()

    idxload(0, sv0, is0)
    wait_idx(0, sv0, is0)
    gather(sv0, rows0, gs0)
    idxload(1, sv1, is1)

    def pair(j, carry):
        a = j * 2
        b = a + 1
        wait_gather(sv0, rows0, gs0)
        idxload(a + 2, sv0, is0)
        scatter(a, rows0, ss0)

        @pl.when(j > 0)
        def _():
            wait_scatter(a - 1, rows1, ss1)

        wait_idx(b, sv1, is1)
        gather(sv1, rows1, gs1)
        wait_gather(sv1, rows1, gs1)

        @pl.when(j < PAIRS - 1)
        def _():
            idxload(b + 2, sv1, is1)

        scatter(b, rows1, ss1)
        wait_scatter(a, rows0, ss0)
        wait_idx(a + 2, sv0, is0)
        gather(sv0, rows0, gs0)
        return carry

    lax.fori_loop(0, PAIRS, pair, 0)
    wait_gather(sv0, rows0, gs0)
    scatter(TAIL, rows0, ss0)
    wait_scatter(TAIL - 1, rows1, ss1)
    wait_scatter(TAIL, rows0, ss0)

    plsc.subcore_barrier()
    pltpu.sync_copy(accx.at[pl.ds(r0, RPT)], out_hbm.at[c, pl.ds(r0, RPT)])


_sc_segsum_rows = functools.partial(
    pl.kernel,
    out_type=jax.ShapeDtypeStruct((NSC, NACC, D), jnp.float32),
    mesh=_MESH,
    scratch_types=[
        pltpu.VMEM((K,), jnp.int32),
        pltpu.VMEM((K,), jnp.int32),
        pltpu.VMEM((NCHUNK, K), jnp.int32),
        pltpu.VMEM((K, D), jnp.float32),
        pltpu.VMEM((K, D), jnp.float32),
        pltpu.VMEM_SHARED((NACC, D), jnp.float32),
        pltpu.SemaphoreType.DMA,
        pltpu.SemaphoreType.DMA,
        pltpu.SemaphoreType.DMA,
        pltpu.SemaphoreType.DMA,
        pltpu.SemaphoreType.DMA,
        pltpu.SemaphoreType.DMA,
    ],
    name="sc_segsum_rows")(_segsum_rows_body)


def _segsum_ea_body(ea_hbm, dst3_hbm, zx_hbm, out_hbm,
                    dst2v, ev0, ev1, rows0, rows1, acce, es0, es1, ss0, ss1):
    c = lax.axis_index("c")
    s = lax.axis_index("s")
    wid = c * NTILE + s
    r0 = pl.multiple_of(s * RPT, RPT)
    pltpu.sync_copy(zx_hbm.at[pl.ds(r0, RPT)], acce.at[pl.ds(r0, RPT)])
    pltpu.sync_copy(zx_hbm.at[pl.ds(0, K)], rows0)
    pltpu.sync_copy(zx_hbm.at[pl.ds(0, K)], rows1)
    pltpu.sync_copy(dst3_hbm.at[wid], dst2v)
    plsc.subcore_barrier()

    e0 = wid * EPT * 4

    def ealoadd(i, buf, sem):
        base = pl.multiple_of(e0 + i * (K * 4), 8)
        pltpu.async_copy(ea_hbm.at[pl.ds(base, K * 4)],
                         buf.at[pl.ds(0, K * 4)], sem)

    def wait_ea(i, buf, sem):
        base = pl.multiple_of(e0 + i * (K * 4), 8)
        pltpu.make_async_copy(ea_hbm.at[pl.ds(base, K * 4)],
                              buf.at[pl.ds(0, K * 4)], sem).wait()

    lanes = lax.iota(jnp.int32, 16)
    is_attr = lanes < 4
    one_col = jnp.where(lanes == 4, 1.0, 0.0).astype(jnp.float32)

    def compose(ebuf, buf):
        for e in range(K):
            window = ebuf[pl.ds(4 * e, 16)]
            buf[e, pl.ds(0, 16)] = jnp.where(is_attr, window, one_col)

    def scatter(i, buf, sem):
        pltpu.async_copy(buf, acce.at[dst2v.at[i]], sem, add=True)

    def wait_scatter(i, buf, sem):
        pltpu.make_async_copy(buf, acce.at[dst2v.at[i]], sem).wait()

    ealoadd(0, ev0, es0)
    wait_ea(0, ev0, es0)
    compose(ev0, rows0)
    scatter(0, rows0, ss0)
    ealoadd(1, ev1, es1)
    ealoadd(2, ev0, es0)

    def pair(j, carry):
        b = j * 2 + 1

        @pl.when(j > 0)
        def _():
            wait_scatter(b - 2, rows1, ss1)

        wait_ea(b, ev1, es1)
        compose(ev1, rows1)
        ealoadd(b + 2, ev1, es1)
        scatter(b, rows1, ss1)
        wait_scatter(b - 1, rows0, ss0)
        wait_ea(b + 1, ev0, es0)
        compose(ev0, rows0)

        @pl.when(j < PAIRS - 2)
        def _():
            ealoadd(b + 3, ev0, es0)

        scatter(b + 1, rows0, ss0)
        return carry

    lax.fori_loop(0, PAIRS - 1, pair, 0)
    wait_scatter(TAIL - 3, rows1, ss1)
    wait_ea(TAIL - 1, ev1, es1)
    compose(ev1, rows1)
    scatter(TAIL - 1, rows1, ss1)
    ealoadd(TAIL, ev0, es0)
    wait_scatter(TAIL - 2, rows0, ss0)
    wait_ea(TAIL, ev0, es0)
    compose(ev0, rows0)
    scatter(TAIL, rows0, ss0)
    wait_scatter(TAIL - 1, rows1, ss1)
    wait_scatter(TAIL, rows0, ss0)

    plsc.subcore_barrier()
    pltpu.sync_copy(acce.at[pl.ds(r0, RPT)], out_hbm.at[c, pl.ds(r0, RPT)])


_sc_segsum_ea = functools.partial(
    pl.kernel,
    out_type=jax.ShapeDtypeStruct((NSC, NACC, D), jnp.float32),
    mesh=_MESH,
    scratch_types=[
        pltpu.VMEM((NCHUNK, K), jnp.int32),
        pltpu.VMEM((K * 4 + 16,), jnp.float32),
        pltpu.VMEM((K * 4 + 16,), jnp.float32),
        pltpu.VMEM((K, D), jnp.float32),
        pltpu.VMEM((K, D), jnp.float32),
        pltpu.VMEM_SHARED((NACC, D), jnp.float32),
        pltpu.SemaphoreType.DMA,
        pltpu.SemaphoreType.DMA,
        pltpu.SemaphoreType.DMA,
        pltpu.SemaphoreType.DMA,
    ],
    name="sc_segsum_ea")(_segsum_ea_body)


_RB = 1024
_GRID = NACC // _RB


def _tc_hidden_body(p_ref, q_ref, wx_ref, we_ref, o_ref):
    p = p_ref[0] + p_ref[1]
    q = q_ref[0] + q_ref[1]
    h = jnp.dot(p, wx_ref[...], preferred_element_type=jnp.float32,
                precision=lax.Precision.HIGHEST)
    h = h + jnp.dot(q, we_ref[...], preferred_element_type=jnp.float32,
                    precision=lax.Precision.HIGHEST)
    o_ref[...] = jnp.maximum(h, 0.0)


def _tc_hidden(P, Q, Wx, We):
    return pl.pallas_call(
        _tc_hidden_body,
        grid=(_GRID,),
        in_specs=[pl.BlockSpec((NSC, _RB, D), lambda i: (0, i, 0)),
                  pl.BlockSpec((NSC, _RB, D), lambda i: (0, i, 0)),
                  pl.BlockSpec((D, D), lambda i: (0, 0)),
                  pl.BlockSpec((D, D), lambda i: (0, 0))],
        out_specs=pl.BlockSpec((_RB, D), lambda i: (i, 0)),
        out_shape=jax.ShapeDtypeStruct((NACC, D), jnp.float32),
    )(P, Q, Wx, We)


def _tc_final_body(p_ref, q_ref, wx_ref, we_ref, wo_ref, bo_ref, o_ref):
    p = p_ref[0] + p_ref[1]
    q = q_ref[0] + q_ref[1]
    h = jnp.dot(p, wx_ref[...], preferred_element_type=jnp.float32,
                precision=lax.Precision.HIGHEST)
    h = h + jnp.dot(q, we_ref[...], preferred_element_type=jnp.float32,
                    precision=lax.Precision.HIGHEST)
    h = jnp.maximum(h, 0.0)
    o_ref[...] = jnp.dot(h, wo_ref[...], preferred_element_type=jnp.float32,
                         precision=lax.Precision.HIGHEST) + bo_ref[...]


def _tc_final(P, Q, Wx, We, Wout, bout):
    return pl.pallas_call(
        _tc_final_body,
        grid=(_GRID,),
        in_specs=[pl.BlockSpec((NSC, _RB, D), lambda i: (0, i, 0)),
                  pl.BlockSpec((NSC, _RB, D), lambda i: (0, i, 0)),
                  pl.BlockSpec((D, D), lambda i: (0, 0)),
                  pl.BlockSpec((D, D), lambda i: (0, 0)),
                  pl.BlockSpec((D, D), lambda i: (0, 0)),
                  pl.BlockSpec((1, D), lambda i: (0, 0))],
        out_specs=pl.BlockSpec((_RB, D), lambda i: (i, 0)),
        out_shape=jax.ShapeDtypeStruct((NACC, D), jnp.float32),
    )(P, Q, Wx, We, Wout, bout)


def _pad_we(We4, b):
    return jnp.concatenate(
        [We4, b[None, :], jnp.zeros((D - 5, D), jnp.float32)], axis=0)


def kernel(x, edge_index, edge_attr, W0, b0, W1, b1, W_out, b_out):
    src = edge_index[0].astype(jnp.int32)
    dst = edge_index[1].astype(jnp.int32)
    dst3 = dst.reshape(NW, NCHUNK, K)
    ea_flat = edge_attr.astype(jnp.float32).reshape(E * 4)
    W0x, W0e = W0[:D], _pad_we(W0[D:], b0)
    W1x, W1e = W1[:D], _pad_we(W1[D:], b1)
    zx = jnp.zeros((NACC, D), jnp.float32)

    P = _sc_segsum_rows(x, src, dst3, zx)
    Q = _sc_segsum_ea(ea_flat, dst3, zx)
    h1 = _tc_hidden(P, Q, W0x, W0e)
    R = _sc_segsum_rows(h1, src, dst3, zx)
    return _tc_final(R, Q, W1x, W1e, W_out, b_out[None, :])[:N]


# speedup vs baseline: 6.6520x; 1.0632x over previous
import functools

import jax
import jax.numpy as jnp
from jax import lax
from jax.experimental import pallas as pl
from jax.experimental.pallas import tpu as pltpu
from jax.experimental.pallas import tpu_sc as plsc

N = 10000
E = 320000
D = 128
NSC = 2
NTILE = 16
NW = NSC * NTILE
EPT = E // NW
K = 80
NCHUNK = EPT // K
PAIRS = NCHUNK // 2
TAIL = NCHUNK - 1
NACC = 10240
RPT = NACC // NTILE

_MESH = plsc.VectorSubcoreMesh(core_axis_name="c", subcore_axis_name="s")


def _segsum_rows_body(x_hbm, src_hbm, dst3_hbm, zx_hbm, out_hbm,
                      sv0, sv1, sv2, dst2v, rows0, rows1, rows2, accx,
                      is0, is1, is2, gs0, gs1, gs2, ss0, ss1, ss2):
    c = lax.axis_index("c")
    s = lax.axis_index("s")
    wid = c * NTILE + s
    r0 = pl.multiple_of(s * RPT, RPT)
    pltpu.sync_copy(zx_hbm.at[pl.ds(r0, RPT)], accx.at[pl.ds(r0, RPT)])
    pltpu.sync_copy(dst3_hbm.at[wid], dst2v)
    plsc.subcore_barrier()

    e0 = wid * EPT
    sv = (sv0, sv1, sv2)
    rows = (rows0, rows1, rows2)
    isem = (is0, is1, is2)
    gsem = (gs0, gs1, gs2)
    ssem = (ss0, ss1, ss2)

    def idxload(i, slot):
        base = pl.multiple_of(e0 + i * K, 8)
        pltpu.async_copy(src_hbm.at[pl.ds(base, K)], sv[slot], isem[slot])

    def wait_idx(i, slot):
        base = pl.multiple_of(e0 + i * K, 8)
        pltpu.make_async_copy(src_hbm.at[pl.ds(base, K)], sv[slot],
                              isem[slot]).wait()

    def gather(slot):
        pltpu.async_copy(x_hbm.at[sv[slot]], rows[slot], gsem[slot])

    def wait_gather(slot):
        pltpu.make_async_copy(x_hbm.at[sv[slot]], rows[slot],
                              gsem[slot]).wait()

    def scatter(i, slot):
        pltpu.async_copy(rows[slot], accx.at[dst2v.at[i]], ssem[slot],
                         add=True)

    def wait_scatter(i, slot):
        pltpu.make_async_copy(rows[slot], accx.at[dst2v.at[i]],
                              ssem[slot]).wait()

    def step(i, slot, do_ws, do_il, do_g):
        wait_gather(slot)
        if do_il:
            idxload(i + 3, slot)
        scatter(i, slot)
        s2 = (i + 2) % 3
        if do_ws:
            wait_scatter(i - 1, s2)
        if do_g:
            wait_idx(i + 2, s2)
            gather(s2)

    idxload(0, 0)
    idxload(1, 1)
    idxload(2, 2)
    wait_idx(0, 0)
    gather(0)
    wait_idx(1, 1)
    gather(1)

    def dyn_step(i, slot):
        wait_gather(slot)
        idxload(i + 3, slot)
        scatter(i, slot)
        s2 = (slot + 2) % 3
        wait_scatter(i - 1, s2)
        wait_idx(i + 2, s2)
        gather(s2)

    step(0, 0, False, True, True)

    def triple(q, carry):
        i = q * 3 + 1
        dyn_step(i, 1)
        dyn_step(i + 1, 2)
        dyn_step(i + 2, 0)
        return carry

    lax.fori_loop(0, (NCHUNK - 4) // 3, triple, 0)
    step(TAIL - 3, 1, True, True, True)
    step(TAIL - 2, 2, True, False, True)
    step(TAIL - 1, 0, True, False, False)
    step(TAIL, 1, True, False, False)
    wait_scatter(TAIL, 1)

    plsc.subcore_barrier()
    pltpu.sync_copy(accx.at[pl.ds(r0, RPT)], out_hbm.at[c, pl.ds(r0, RPT)])


_sc_segsum_rows = functools.partial(
    pl.kernel,
    out_type=jax.ShapeDtypeStruct((NSC, NACC, D), jnp.float32),
    mesh=_MESH,
    scratch_types=[
        pltpu.VMEM((K,), jnp.int32),
        pltpu.VMEM((K,), jnp.int32),
        pltpu.VMEM((K,), jnp.int32),
        pltpu.VMEM((NCHUNK, K), jnp.int32),
        pltpu.VMEM((K, D), jnp.float32),
        pltpu.VMEM((K, D), jnp.float32),
        pltpu.VMEM((K, D), jnp.float32),
        pltpu.VMEM_SHARED((NACC, D), jnp.float32),
        pltpu.SemaphoreType.DMA,
        pltpu.SemaphoreType.DMA,
        pltpu.SemaphoreType.DMA,
        pltpu.SemaphoreType.DMA,
        pltpu.SemaphoreType.DMA,
        pltpu.SemaphoreType.DMA,
        pltpu.SemaphoreType.DMA,
        pltpu.SemaphoreType.DMA,
        pltpu.SemaphoreType.DMA,
    ],
    name="sc_segsum_rows")(_segsum_rows_body)


def _segsum_ea_body(ea_hbm, dst3_hbm, zx_hbm, out_hbm,
                    dst2v, ev0, ev1, rows0, rows1, acce, es0, es1, ss0, ss1):
    c = lax.axis_index("c")
    s = lax.axis_index("s")
    wid = c * NTILE + s
    r0 = pl.multiple_of(s * RPT, RPT)
    pltpu.sync_copy(zx_hbm.at[pl.ds(r0, RPT)], acce.at[pl.ds(r0, RPT)])
    pltpu.sync_copy(zx_hbm.at[pl.ds(0, K)], rows0)
    pltpu.sync_copy(zx_hbm.at[pl.ds(0, K)], rows1)
    pltpu.sync_copy(dst3_hbm.at[wid], dst2v)
    plsc.subcore_barrier()

    e0 = wid * EPT * 4

    def ealoadd(i, buf, sem):
        base = pl.multiple_of(e0 + i * (K * 4), 8)
        pltpu.async_copy(ea_hbm.at[pl.ds(base, K * 4)],
                         buf.at[pl.ds(0, K * 4)], sem)

    def wait_ea(i, buf, sem):
        base = pl.multiple_of(e0 + i * (K * 4), 8)
        pltpu.make_async_copy(ea_hbm.at[pl.ds(base, K * 4)],
                              buf.at[pl.ds(0, K * 4)], sem).wait()

    lanes = lax.iota(jnp.int32, 16)
    is_attr = lanes < 4
    one_col = jnp.where(lanes == 4, 1.0, 0.0).astype(jnp.float32)

    def compose(ebuf, buf):
        for e in range(K):
            window = ebuf[pl.ds(4 * e, 16)]
            buf[e, pl.ds(0, 16)] = jnp.where(is_attr, window, one_col)

    def scatter(i, buf, sem):
        pltpu.async_copy(buf, acce.at[dst2v.at[i]], sem, add=True)

    def wait_scatter(i, buf, sem):
        pltpu.make_async_copy(buf, acce.at[dst2v.at[i]], sem).wait()

    ealoadd(0, ev0, es0)
    wait_ea(0, ev0, es0)
    compose(ev0, rows0)
    scatter(0, rows0, ss0)
    ealoadd(1, ev1, es1)
    ealoadd(2, ev0, es0)

    def pair(j, carry):
        b = j * 2 + 1

        @pl.when(j > 0)
        def _():
            wait_scatter(b - 2, rows1, ss1)

        wait_ea(b, ev1, es1)
        compose(ev1, rows1)
        ealoadd(b + 2, ev1, es1)
        scatter(b, rows1, ss1)
        wait_scatter(b - 1, rows0, ss0)
        wait_ea(b + 1, ev0, es0)
        compose(ev0, rows0)

        @pl.when(j < PAIRS - 2)
        def _():
            ealoadd(b + 3, ev0, es0)

        scatter(b + 1, rows0, ss0)
        return carry

    lax.fori_loop(0, PAIRS - 1, pair, 0)
    wait_scatter(TAIL - 3, rows1, ss1)
    wait_ea(TAIL - 1, ev1, es1)
    compose(ev1, rows1)
    scatter(TAIL - 1, rows1, ss1)
    ealoadd(TAIL, ev0, es0)
    wait_scatter(TAIL - 2, rows0, ss0)
    wait_ea(TAIL, ev0, es0)
    compose(ev0, rows0)
    scatter(TAIL, rows0, ss0)
    wait_scatter(TAIL - 1, rows1, ss1)
    wait_scatter(TAIL, rows0, ss0)

    plsc.subcore_barrier()
    pltpu.sync_copy(acce.at[pl.ds(r0, RPT)], out_hbm.at[c, pl.ds(r0, RPT)])


_sc_segsum_ea = functools.partial(
    pl.kernel,
    out_type=jax.ShapeDtypeStruct((NSC, NACC, D), jnp.float32),
    mesh=_MESH,
    scratch_types=[
        pltpu.VMEM((NCHUNK, K), jnp.int32),
        pltpu.VMEM((K * 4 + 16,), jnp.float32),
        pltpu.VMEM((K * 4 + 16,), jnp.float32),
        pltpu.VMEM((K, D), jnp.float32),
        pltpu.VMEM((K, D), jnp.float32),
        pltpu.VMEM_SHARED((NACC, D), jnp.float32),
        pltpu.SemaphoreType.DMA,
        pltpu.SemaphoreType.DMA,
        pltpu.SemaphoreType.DMA,
        pltpu.SemaphoreType.DMA,
    ],
    name="sc_segsum_ea")(_segsum_ea_body)


_RB = 1024
_GRID = NACC // _RB


def _tc_hidden_body(p_ref, q_ref, wx_ref, we_ref, o_ref):
    p = p_ref[0] + p_ref[1]
    q = q_ref[0] + q_ref[1]
    h = jnp.dot(p, wx_ref[...], preferred_element_type=jnp.float32,
                precision=lax.Precision.HIGHEST)
    h = h + jnp.dot(q, we_ref[...], preferred_element_type=jnp.float32,
                    precision=lax.Precision.HIGHEST)
    o_ref[...] = jnp.maximum(h, 0.0)


def _tc_hidden(P, Q, Wx, We):
    return pl.pallas_call(
        _tc_hidden_body,
        grid=(_GRID,),
        in_specs=[pl.BlockSpec((NSC, _RB, D), lambda i: (0, i, 0)),
                  pl.BlockSpec((NSC, _RB, D), lambda i: (0, i, 0)),
                  pl.BlockSpec((D, D), lambda i: (0, 0)),
                  pl.BlockSpec((D, D), lambda i: (0, 0))],
        out_specs=pl.BlockSpec((_RB, D), lambda i: (i, 0)),
        out_shape=jax.ShapeDtypeStruct((NACC, D), jnp.float32),
    )(P, Q, Wx, We)


def _tc_final_body(p_ref, q_ref, wx_ref, we_ref, wo_ref, bo_ref, o_ref):
    p = p_ref[0] + p_ref[1]
    q = q_ref[0] + q_ref[1]
    h = jnp.dot(p, wx_ref[...], preferred_element_type=jnp.float32,
                precision=lax.Precision.HIGHEST)
    h = h + jnp.dot(q, we_ref[...], preferred_element_type=jnp.float32,
                    precision=lax.Precision.HIGHEST)
    h = jnp.maximum(h, 0.0)
    o_ref[...] = jnp.dot(h, wo_ref[...], preferred_element_type=jnp.float32,
                         precision=lax.Precision.HIGHEST) + bo_ref[...]


def _tc_final(P, Q, Wx, We, Wout, bout):
    rb = N // _GRID
    return pl.pallas_call(
        _tc_final_body,
        grid=(_GRID,),
        in_specs=[pl.BlockSpec((NSC, rb, D), lambda i: (0, i, 0)),
                  pl.BlockSpec((NSC, rb, D), lambda i: (0, i, 0)),
                  pl.BlockSpec((D, D), lambda i: (0, 0)),
                  pl.BlockSpec((D, D), lambda i: (0, 0)),
                  pl.BlockSpec((D, D), lambda i: (0, 0)),
                  pl.BlockSpec((1, D), lambda i: (0, 0))],
        out_specs=pl.BlockSpec((rb, D), lambda i: (i, 0)),
        out_shape=jax.ShapeDtypeStruct((N, D), jnp.float32),
    )(P, Q, Wx, We, Wout, bout)


def _pad_we(We4, b):
    return jnp.concatenate(
        [We4, b[None, :], jnp.zeros((D - 5, D), jnp.float32)], axis=0)


def kernel(x, edge_index, edge_attr, W0, b0, W1, b1, W_out, b_out):
    src = edge_index[0].astype(jnp.int32)
    dst = edge_index[1].astype(jnp.int32)
    dst3 = dst.reshape(NW, NCHUNK, K)
    ea_flat = edge_attr.astype(jnp.float32).reshape(E * 4)
    W0x, W0e = W0[:D], _pad_we(W0[D:], b0)
    W1x, W1e = W1[:D], _pad_we(W1[D:], b1)
    zx = jnp.zeros((NACC, D), jnp.float32)

    P = _sc_segsum_rows(x, src, dst3, zx)
    Q = _sc_segsum_ea(ea_flat, dst3, zx)
    h1 = _tc_hidden(P, Q, W0x, W0e)
    R = _sc_segsum_rows(h1, src, dst3, zx)
    return _tc_final(R, Q, W1x, W1e, W_out, b_out[None, :])
